# Initial kernel scaffold; baseline (speedup 1.0000x reference)
import jax
import jax.numpy as jnp
from jax.experimental import pallas as pl


def kernel(locations, table):
    raise NotImplementedError("write your pallas kernel here")


# speedup vs baseline: 1.6454x; 1.6454x over previous
import functools

import jax
import jax.numpy as jnp
from jax import lax
from jax.experimental import pallas as pl
from jax.experimental.pallas import tpu as pltpu
from jax.experimental.pallas import tpu_sc as plsc


def _broadcast_copy(table, B):
    L, D = table.shape
    info = plsc.get_sparse_core_info()
    NC, NS = info.num_cores, info.num_subcores
    NW = NC * NS
    rows_per_w = L // NW
    chunk = 64
    nchunk = rows_per_w // chunk

    mesh = plsc.VectorSubcoreMesh(core_axis_name="c", subcore_axis_name="s")

    @functools.partial(
        pl.kernel,
        mesh=mesh,
        out_type=jax.ShapeDtypeStruct((B * L, D), jnp.float32),
        scratch_types=[
            pltpu.VMEM((chunk, D), jnp.float32),
            pltpu.SemaphoreType.DMA,
        ],
    )
    def k(table_hbm, out_hbm, buf, sem):
        wid = lax.axis_index("s") * NC + lax.axis_index("c")
        base = wid * rows_per_w
        for c in range(nchunk):
            r = base + c * chunk
            pltpu.async_copy(table_hbm.at[pl.ds(r, chunk), :], buf, sem).wait()
            for b in range(B):
                pltpu.sync_copy(buf, out_hbm.at[pl.ds(b * L + r, chunk), :])

    return k(table).reshape(B, L, D)


def kernel(locations, table):
    B = locations.shape[0]
    return _broadcast_copy(table, B)
